# baseline (speedup 1.0000x reference)
import jax
import jax.numpy as jnp
from jax import lax
from jax.experimental import pallas as pl
from jax.experimental.pallas import tpu as pltpu

_THRESHOLD = 0.5
_NEG_POS_RATIO = 3
_ALPHA = 10.0
_F32_INF_BITS = 0x7F800000


def _match_body(boxes_ref, boxes_t_ref, labels_ref, pxy_ref, pcx_ref,
                pred_ref, part_ref, ce_ref):
    boxes = boxes_ref[0]
    boxes_t = boxes_t_ref[0]
    labels = labels_ref[0]
    pxy = pxy_ref[...]
    pcx = pcx_ref[...]
    pred = pred_ref[0]
    x = pred[0:-4, :]
    ploc = pred[-4:, :]

    n_obj, _ = boxes.shape
    C, P = x.shape

    bx0, by0, bx1, by1 = (boxes[:, i:i + 1] for i in range(4))
    px0, py0, px1, py1 = (pxy[i:i + 1, :] for i in range(4))
    iw = jnp.maximum(jnp.minimum(bx1, px1) - jnp.maximum(bx0, px0), 0.0)
    ih = jnp.maximum(jnp.minimum(by1, py1) - jnp.maximum(by0, py0), 0.0)
    inter = iw * ih
    a_box = (bx1 - bx0) * (by1 - by0)
    a_pri = (px1 - px0) * (py1 - py0)
    iou = inter / (a_box + a_pri - inter)

    lane = lax.broadcasted_iota(jnp.int32, (n_obj, P), 1)
    sub = lax.broadcasted_iota(jnp.int32, (n_obj, P), 0)

    row_max = jnp.max(iou, axis=1, keepdims=True)
    obj_idx = jnp.min(jnp.where(iou == row_max, lane, P), axis=1,
                      keepdims=True)
    iou2 = jnp.where(lane == obj_idx, 1.0, iou)

    col_max = jnp.max(iou2, axis=0, keepdims=True)
    pos = col_max >= _THRESHOLD
    mpi = jnp.min(jnp.where(iou2 == col_max, sub, n_obj), axis=0,
                  keepdims=True)

    oh = (sub == mpi).astype(jnp.float32)
    a_mat = jnp.concatenate(
        [boxes_t, labels.astype(jnp.float32),
         jnp.zeros((3, n_obj), jnp.float32)], axis=0)
    sel = lax.dot_general(a_mat, oh, (((1,), (0,)), ((), ())),
                          preferred_element_type=jnp.float32)
    sx0, sy0, sx1, sy1, slab = (sel[i:i + 1, :] for i in range(5))

    pw = pcx[2:3, :]
    ph = pcx[3:4, :]
    g0 = (sx0 + sx1 - 2.0 * pcx[0:1, :]) * 5.0 / pw
    g1 = (sy0 + sy1 - 2.0 * pcx[1:2, :]) * 5.0 / ph
    g2 = 5.0 * jnp.log((sx1 - sx0) / pw)
    g3 = 5.0 * jnp.log((sy1 - sy0) / ph)

    loc_abs = (jnp.abs(ploc[0:1, :] - g0) + jnp.abs(ploc[1:2, :] - g1)
               + jnp.abs(ploc[2:3, :] - g2) + jnp.abs(ploc[3:4, :] - g3))
    loc_sum = jnp.sum(jnp.where(pos, loc_abs, 0.0))

    m = jnp.max(x, axis=0, keepdims=True)
    e = jnp.exp(x - m)
    ones8 = jnp.ones((8, C), jnp.float32)
    se = lax.dot_general(ones8, e, (((1,), (0,)), ((), ())),
                         preferred_element_type=jnp.float32)
    lse = m + jnp.log(se[0:1, :])
    ce0 = lse - x[0:1, :]

    tc_i = jnp.where(pos, slab.astype(jnp.int32), 0)
    cls_iota = lax.broadcasted_iota(jnp.int32, (C, P), 0)
    mask_pc = (cls_iota == tc_i) & pos
    sum_xtc = jnp.sum(jnp.where(mask_pc, x, 0.0))
    sum_lse_pos = jnp.sum(jnp.where(pos, lse, 0.0))
    conf_pos = sum_lse_pos - sum_xtc

    n_pos = jnp.sum(pos.astype(jnp.int32))
    ce_ref[...] = jnp.where(pos, 0.0, ce0)[None]

    part_ref[0, 0, 0] = loc_sum
    part_ref[0, 0, 1] = n_pos.astype(jnp.float32)
    part_ref[0, 0, 2] = conf_pos
    part_ref[0, 0, 3] = 0.0


def _topk_body(ce_ref, parts_ref, out_ref):
    v = ce_ref[...]
    B, P = v.shape
    parts = parts_ref[...]
    n_pos_f = parts[:, 1:2]
    k = jnp.minimum(n_pos_f.astype(jnp.int32) * _NEG_POS_RATIO,
                    P)
    k_f = k.astype(jnp.float32)
    vbits = lax.bitcast_convert_type(v, jnp.int32)
    ones_p = jnp.ones((P, 8), jnp.float32)

    def row_sum(a):
        r = lax.dot_general(a, ones_p, (((1,), (0,)), ((), ())),
                            preferred_element_type=jnp.float32)
        return r[:, 0:1]

    def bs_body(_, carry):
        lo, hi = carry
        mid = lo + lax.div(hi - lo, 2)
        cnt = row_sum(jnp.where(vbits > mid, 1.0, 0.0))
        take_hi = cnt < k_f
        return (jnp.where(take_hi, lo, mid + 1),
                jnp.where(take_hi, mid, hi))

    t_bits, _ = lax.fori_loop(
        0, 31, bs_body,
        (jnp.zeros((B, 1), jnp.int32),
         jnp.full((B, 1), _F32_INF_BITS, jnp.int32)))
    t_val = lax.bitcast_convert_type(t_bits, jnp.float32)
    gt = vbits > t_bits
    s_gt = row_sum(jnp.where(gt, v, 0.0))
    c_gt = row_sum(jnp.where(gt, 1.0, 0.0))
    hard = jnp.where(k > 0, s_gt + (k_f - c_gt) * t_val, 0.0)

    conf_hard_sum = jnp.sum(hard)
    loc_sum = jnp.sum(parts[:, 0:1])
    n_pos_sum = jnp.sum(n_pos_f)
    conf_pos_sum = jnp.sum(parts[:, 2:3])
    loc_loss = _ALPHA * loc_sum / (n_pos_sum * 4.0)
    conf_loss = (conf_hard_sum + conf_pos_sum) / n_pos_sum
    out_ref[0, 0] = conf_loss + loc_loss
    out_ref[0, 1] = loc_loss
    out_ref[0, 2] = conf_loss


def kernel(pred_loc, pred_cls, b_boxes, b_labels, priors_cxcy):
    B, P, C = pred_cls.shape
    n_obj = b_boxes.shape[1]

    priors_xy = jnp.concatenate(
        [priors_cxcy[:, :2] - priors_cxcy[:, 2:] / 2.0,
         priors_cxcy[:, :2] + priors_cxcy[:, 2:] / 2.0], axis=1)
    pxy_t = priors_xy.T
    pcx_t = priors_cxcy.T
    boxes_t = jnp.transpose(b_boxes, (0, 2, 1))
    labels3 = b_labels.astype(jnp.int32)[:, None, :]

    n_chunks = 1
    G = B // n_chunks
    match_call = pl.pallas_call(
        _match_body,
        grid=(G,),
        in_specs=[
            pl.BlockSpec((1, n_obj, 4), lambda b: (b, 0, 0)),
            pl.BlockSpec((1, 4, n_obj), lambda b: (b, 0, 0)),
            pl.BlockSpec((1, 1, n_obj), lambda b: (b, 0, 0)),
            pl.BlockSpec((4, P), lambda b: (0, 0)),
            pl.BlockSpec((4, P), lambda b: (0, 0)),
            pl.BlockSpec((1, C + 4, P), lambda b: (b, 0, 0)),
        ],
        out_specs=[
            pl.BlockSpec((1, 1, 4), lambda b: (b, 0, 0),
                         memory_space=pltpu.SMEM),
            pl.BlockSpec((1, 1, P), lambda b: (b, 0, 0)),
        ],
        out_shape=[
            jax.ShapeDtypeStruct((G, 1, 4), jnp.float32),
            jax.ShapeDtypeStruct((G, 1, P), jnp.float32),
        ],
        compiler_params=pltpu.CompilerParams(
            dimension_semantics=("parallel",)),
    )
    pred_all = jnp.concatenate([pred_cls, pred_loc], axis=2)
    parts_chunks, ce_chunks = [], []
    for i in range(n_chunks):
        sl = slice(i * G, (i + 1) * G)
        pred_t = jnp.transpose(pred_all[sl], (0, 2, 1))
        p_i, ce_i = match_call(b_boxes[sl], boxes_t[sl], labels3[sl],
                               pxy_t, pcx_t, pred_t)
        parts_chunks.append(p_i)
        ce_chunks.append(ce_i.reshape(G, P))

    parts = jnp.concatenate(parts_chunks, axis=0)
    ce_neg = (ce_chunks[0] if n_chunks == 1
              else jnp.concatenate(ce_chunks, axis=0))
    res = pl.pallas_call(
        _topk_body,
        out_specs=pl.BlockSpec(memory_space=pltpu.SMEM),
        out_shape=jax.ShapeDtypeStruct((1, 3), jnp.float32),
    )(ce_neg, parts.reshape(B, 4))
    return res[0, 0], res[0, 1], res[0, 2]


# speedup vs baseline: 1.1775x; 1.1775x over previous
import jax
import jax.numpy as jnp
from jax import lax
from jax.experimental import pallas as pl
from jax.experimental.pallas import tpu as pltpu

_THRESHOLD = 0.5
_NEG_POS_RATIO = 3
_ALPHA = 10.0
_F32_INF_BITS = 0x7F800000


def _match_body(boxes_ref, boxes_t_ref, labels_ref, pxy_ref, pcx_ref,
                ploc_ref, pcls_ref, part_ref, ce_ref):
    boxes = boxes_ref[0]
    boxes_t = boxes_t_ref[0]
    labels = labels_ref[0]
    pxy = pxy_ref[...]
    pcx = pcx_ref[...]
    ploc = ploc_ref[0]
    x = pcls_ref[0]

    n_obj, _ = boxes.shape
    C, P = x.shape

    bx0, by0, bx1, by1 = (boxes[:, i:i + 1] for i in range(4))
    px0, py0, px1, py1 = (pxy[i:i + 1, :] for i in range(4))
    iw = jnp.maximum(jnp.minimum(bx1, px1) - jnp.maximum(bx0, px0), 0.0)
    ih = jnp.maximum(jnp.minimum(by1, py1) - jnp.maximum(by0, py0), 0.0)
    inter = iw * ih
    a_box = (bx1 - bx0) * (by1 - by0)
    a_pri = (px1 - px0) * (py1 - py0)
    iou = inter / (a_box + a_pri - inter)

    lane = lax.broadcasted_iota(jnp.int32, (n_obj, P), 1)
    sub = lax.broadcasted_iota(jnp.int32, (n_obj, P), 0)

    row_max = jnp.max(iou, axis=1, keepdims=True)
    obj_idx = jnp.min(jnp.where(iou == row_max, lane, P), axis=1,
                      keepdims=True)
    iou2 = jnp.where(lane == obj_idx, 1.0, iou)

    col_max = jnp.max(iou2, axis=0, keepdims=True)
    pos = col_max >= _THRESHOLD
    mpi = jnp.min(jnp.where(iou2 == col_max, sub, n_obj), axis=0,
                  keepdims=True)

    oh = (sub == mpi).astype(jnp.float32)
    a_mat = jnp.concatenate(
        [boxes_t, labels.astype(jnp.float32),
         jnp.zeros((3, n_obj), jnp.float32)], axis=0)
    sel = lax.dot_general(a_mat, oh, (((1,), (0,)), ((), ())),
                          preferred_element_type=jnp.float32)
    sx0, sy0, sx1, sy1, slab = (sel[i:i + 1, :] for i in range(5))

    pw = pcx[2:3, :]
    ph = pcx[3:4, :]
    g0 = (sx0 + sx1 - 2.0 * pcx[0:1, :]) * 5.0 / pw
    g1 = (sy0 + sy1 - 2.0 * pcx[1:2, :]) * 5.0 / ph
    g2 = 5.0 * jnp.log((sx1 - sx0) / pw)
    g3 = 5.0 * jnp.log((sy1 - sy0) / ph)

    loc_abs = (jnp.abs(ploc[0:1, :] - g0) + jnp.abs(ploc[1:2, :] - g1)
               + jnp.abs(ploc[2:3, :] - g2) + jnp.abs(ploc[3:4, :] - g3))
    loc_sum = jnp.sum(jnp.where(pos, loc_abs, 0.0))

    m = jnp.max(x, axis=0, keepdims=True)
    e = jnp.exp(x - m)
    ones8 = jnp.ones((8, C), jnp.float32)
    se = lax.dot_general(ones8, e, (((1,), (0,)), ((), ())),
                         preferred_element_type=jnp.float32)
    lse = m + jnp.log(se[0:1, :])
    ce0 = lse - x[0:1, :]

    tc_i = jnp.where(pos, slab.astype(jnp.int32), 0)
    cls_iota = lax.broadcasted_iota(jnp.int32, (C, P), 0)
    mask_pc = (cls_iota == tc_i) & pos
    sum_xtc = jnp.sum(jnp.where(mask_pc, x, 0.0))
    sum_lse_pos = jnp.sum(jnp.where(pos, lse, 0.0))
    conf_pos = sum_lse_pos - sum_xtc

    n_pos = jnp.sum(pos.astype(jnp.int32))
    ce_ref[...] = jnp.where(pos, 0.0, ce0)[None]

    part_ref[0, 0, 0] = loc_sum
    part_ref[0, 0, 1] = n_pos.astype(jnp.float32)
    part_ref[0, 0, 2] = conf_pos
    part_ref[0, 0, 3] = 0.0


def _topk_body(ce_ref, parts_ref, out_ref):
    v = ce_ref[...]
    B, P = v.shape
    parts = parts_ref[...]
    n_pos_f = parts[:, 1:2]
    k = jnp.minimum(n_pos_f.astype(jnp.int32) * _NEG_POS_RATIO,
                    P)
    k_f = k.astype(jnp.float32)
    vbits = lax.bitcast_convert_type(v, jnp.int32)
    ones_p = jnp.ones((P, 8), jnp.float32)

    def row_sum(a):
        r = lax.dot_general(a, ones_p, (((1,), (0,)), ((), ())),
                            preferred_element_type=jnp.float32)
        return r[:, 0:1]

    def bs_body(_, carry):
        lo, hi = carry
        mid = lo + lax.div(hi - lo, 2)
        cnt = row_sum(jnp.where(vbits > mid, 1.0, 0.0))
        take_hi = cnt < k_f
        return (jnp.where(take_hi, lo, mid + 1),
                jnp.where(take_hi, mid, hi))

    t_bits, _ = lax.fori_loop(
        0, 31, bs_body,
        (jnp.zeros((B, 1), jnp.int32),
         jnp.full((B, 1), _F32_INF_BITS, jnp.int32)))
    t_val = lax.bitcast_convert_type(t_bits, jnp.float32)
    gt = vbits > t_bits
    s_gt = row_sum(jnp.where(gt, v, 0.0))
    c_gt = row_sum(jnp.where(gt, 1.0, 0.0))
    hard = jnp.where(k > 0, s_gt + (k_f - c_gt) * t_val, 0.0)

    conf_hard_sum = jnp.sum(hard)
    loc_sum = jnp.sum(parts[:, 0:1])
    n_pos_sum = jnp.sum(n_pos_f)
    conf_pos_sum = jnp.sum(parts[:, 2:3])
    loc_loss = _ALPHA * loc_sum / (n_pos_sum * 4.0)
    conf_loss = (conf_hard_sum + conf_pos_sum) / n_pos_sum
    out_ref[0, 0] = conf_loss + loc_loss
    out_ref[0, 1] = loc_loss
    out_ref[0, 2] = conf_loss


def kernel(pred_loc, pred_cls, b_boxes, b_labels, priors_cxcy):
    B, P, C = pred_cls.shape
    n_obj = b_boxes.shape[1]

    priors_xy = jnp.concatenate(
        [priors_cxcy[:, :2] - priors_cxcy[:, 2:] / 2.0,
         priors_cxcy[:, :2] + priors_cxcy[:, 2:] / 2.0], axis=1)
    pxy_t = priors_xy.T
    pcx_t = priors_cxcy.T
    boxes_t = jnp.transpose(b_boxes, (0, 2, 1))
    labels3 = b_labels.astype(jnp.int32)[:, None, :]

    n_chunks = 1
    G = B // n_chunks
    match_call = pl.pallas_call(
        _match_body,
        grid=(G,),
        in_specs=[
            pl.BlockSpec((1, n_obj, 4), lambda b: (b, 0, 0)),
            pl.BlockSpec((1, 4, n_obj), lambda b: (b, 0, 0)),
            pl.BlockSpec((1, 1, n_obj), lambda b: (b, 0, 0)),
            pl.BlockSpec((4, P), lambda b: (0, 0)),
            pl.BlockSpec((4, P), lambda b: (0, 0)),
            pl.BlockSpec((1, 4, P), lambda b: (b, 0, 0)),
            pl.BlockSpec((1, C, P), lambda b: (b, 0, 0)),
        ],
        out_specs=[
            pl.BlockSpec((1, 1, 4), lambda b: (b, 0, 0),
                         memory_space=pltpu.SMEM),
            pl.BlockSpec((1, 1, P), lambda b: (b, 0, 0)),
        ],
        out_shape=[
            jax.ShapeDtypeStruct((G, 1, 4), jnp.float32),
            jax.ShapeDtypeStruct((G, 1, P), jnp.float32),
        ],
        compiler_params=pltpu.CompilerParams(
            dimension_semantics=("parallel",)),
    )
    parts_chunks, ce_chunks = [], []
    for i in range(n_chunks):
        sl = slice(i * G, (i + 1) * G)
        ploc_t = jnp.transpose(pred_loc[sl], (0, 2, 1))
        pcls_t = jnp.transpose(pred_cls[sl], (0, 2, 1))
        p_i, ce_i = match_call(b_boxes[sl], boxes_t[sl], labels3[sl],
                               pxy_t, pcx_t, ploc_t, pcls_t)
        parts_chunks.append(p_i)
        ce_chunks.append(ce_i.reshape(G, P))

    parts = jnp.concatenate(parts_chunks, axis=0)
    ce_neg = (ce_chunks[0] if n_chunks == 1
              else jnp.concatenate(ce_chunks, axis=0))
    res = pl.pallas_call(
        _topk_body,
        out_specs=pl.BlockSpec(memory_space=pltpu.SMEM),
        out_shape=jax.ShapeDtypeStruct((1, 3), jnp.float32),
    )(ce_neg, parts.reshape(B, 4))
    return res[0, 0], res[0, 1], res[0, 2]
